# Initial kernel scaffold; baseline (speedup 1.0000x reference)
import jax
import jax.numpy as jnp
from jax.experimental import pallas as pl


def kernel(a, hash_idx, W):
    raise NotImplementedError("write your pallas kernel here")


# speedup vs baseline: 666.0462x; 666.0462x over previous
import jax
import jax.numpy as jnp
from jax.experimental import pallas as pl


def _tc_kernel(a_ref, hm_ref, hb_ref, w_ref, out_ref):
    fo = hm_ref.shape[0]
    wb = jnp.broadcast_to(w_ref[0, :], (fo, w_ref.shape[1]))
    weff = jnp.take_along_axis(wb, hm_ref[:, :], axis=1)
    bias = jnp.take_along_axis(wb, hb_ref[:, :], axis=1)[:, 0]
    acc = jax.lax.dot_general(
        a_ref[:, :], weff,
        dimension_numbers=(((1,), (1,)), ((), ())),
        preferred_element_type=jnp.float32,
    )
    out_ref[:, :] = acc + bias[None, :]


def kernel(a, hash_idx, W):
    B, FI = a.shape
    FO = hash_idx.shape[0]
    K = W.shape[0]
    hash_main = hash_idx[:, :FI]
    hash_bias = hash_idx[:, FI:]
    w2 = W.reshape(1, K)
    return pl.pallas_call(
        _tc_kernel,
        out_shape=jax.ShapeDtypeStruct((B, FO), jnp.float32),
    )(a, hash_main, hash_bias, w2)
